# baseline (speedup 1.0000x reference)
import functools
import math

import jax
import jax.numpy as jnp
from jax import lax
from jax.experimental import pallas as pl
from jax.experimental.pallas import tpu as pltpu
from jax.experimental.pallas import tpu_sc as plsc

_SCALE = 64.0
_COS_M = math.cos(0.5)
_SIN_M = math.sin(0.5)

_NC = 2
_NS = 16
_NW = _NC * _NS
_K = 5


def _rsqrt_newton(x):
    i = plsc.bitcast(x, jnp.int32)
    y = plsc.bitcast(jnp.int32(0x5F3759DF) - (i >> 1), jnp.float32)
    for _ in range(4):
        y = y * (1.5 - 0.5 * x * y * y)
    return y


def _sc_body(logits_hbm, labels_hbm, out_hbm, row_v, lab_v, red_v,
             sems_in, sems_out):
    n, c = logits_hbm.shape
    rows_per = n // _NW
    ch = c // _K
    wid = lax.axis_index("s") * _NC + lax.axis_index("c")
    base = wid * rows_per
    pltpu.sync_copy(labels_hbm.at[pl.ds(base, rows_per)], lab_v)
    lanes = lax.iota(jnp.int32, 16)
    zero = jnp.zeros((16,), jnp.float32)

    def do_row(j, carry):
        row = base + j

        in_copies = {}

        def start_in(k):
            @pl.when(j > 0)
            def _():
                pltpu.make_async_copy(
                    row_v.at[pl.ds(k * ch, ch)],
                    out_hbm.at[row - 1].at[pl.ds(k * ch, ch)],
                    sems_out[k],
                ).wait()
            in_copies[k] = pltpu.async_copy(
                logits_hbm.at[row].at[pl.ds(k * ch, ch)],
                row_v.at[pl.ds(k * ch, ch)],
                sems_in[k],
            )

        start_in(0)
        accs = (zero, zero, zero, zero, zero)
        for k in range(_K):
            if k + 1 < _K:
                start_in(k + 1)
            in_copies[k].wait()

            @plsc.parallel_loop(0, ch, step=80, unroll=5, carry=accs)
            def _sumsq(i, accs_c):
                outs = []
                for u, a in enumerate(accs_c):
                    v = row_v[pl.ds(k * ch + i + u * 16, 16)]
                    outs.append(a + v * v)
                return tuple(outs)

            accs = _sumsq

        a0, a1, a2, a3, a4 = accs
        acc = ((a0 + a1) + (a2 + a3)) + a4
        for s in (1, 2, 4, 8):
            red_v[pl.ds(0, 16)] = acc
            acc = acc + plsc.load_gather(red_v, [lanes ^ s])
        inv = _rsqrt_newton(jnp.maximum(acc, 1e-24))

        lab = plsc.load_gather(lab_v, [jnp.full((16,), j, jnp.int32)])
        validv = lab != -1
        lab_safe = jnp.where(validv, lab, 0)
        t = plsc.load_gather(row_v, [lab_safe]) * inv
        t_clip = jnp.clip(t, -1.0, 1.0)
        s2 = jnp.maximum(1.0 - t_clip * t_clip, 1e-30)
        sin_theta = s2 * _rsqrt_newton(s2)
        with_margin = t_clip * _COS_M - _SIN_M * sin_theta
        new_val = jnp.where(validv, with_margin, t) * _SCALE

        scale = inv * _SCALE

        for k in range(_K):
            @plsc.parallel_loop(0, ch, step=80, unroll=5)
            def _scale(i):
                for u in range(5):
                    sl = pl.ds(k * ch + i + u * 16, 16)
                    row_v[sl] = row_v[sl] * scale

            if k == 0:
                lane0 = lanes == 0
                in_chunk0 = lab_safe < ch
                plsc.store_scatter(row_v, [jnp.where(in_chunk0, lab_safe, 0)],
                                   jnp.where(in_chunk0, new_val,
                                             row_v[pl.ds(0, 16)]),
                                   mask=lane0)
            else:
                lane0 = lanes == 0
                lo = k * ch
                in_chunk = jnp.logical_and(lab_safe >= lo, lab_safe < lo + ch)
                plsc.store_scatter(row_v, [jnp.where(in_chunk, lab_safe, lo)],
                                   jnp.where(in_chunk, new_val,
                                             row_v[pl.ds(lo, 16)]),
                                   mask=lane0)
            pltpu.async_copy(
                row_v.at[pl.ds(k * ch, ch)],
                out_hbm.at[row].at[pl.ds(k * ch, ch)],
                sems_out[k],
            )
        return carry

    lax.fori_loop(0, rows_per, do_row, 0)

    last = base + rows_per - 1
    for k in range(_K):
        pltpu.make_async_copy(
            row_v.at[pl.ds(k * ch, ch)],
            out_hbm.at[last].at[pl.ds(k * ch, ch)],
            sems_out[k],
        ).wait()


@jax.jit
def _run(logits, labels):
    n, c = logits.shape
    mesh = plsc.VectorSubcoreMesh(core_axis_name="c", subcore_axis_name="s")
    return pl.kernel(
        _sc_body,
        out_type=jax.ShapeDtypeStruct((n, c), jnp.float32),
        mesh=mesh,
        scratch_types=[
            pltpu.VMEM((c,), jnp.float32),
            pltpu.VMEM((n // _NW,), jnp.int32),
            pltpu.VMEM((128,), jnp.float32),
            [pltpu.SemaphoreType.DMA] * _K,
            [pltpu.SemaphoreType.DMA] * _K,
        ],
        compiler_params=pltpu.CompilerParams(needs_layout_passes=False, use_tc_tiling_on_sc=False),
    )(logits, labels)


def kernel(logits, labels):
    return _run(logits, labels.astype(jnp.int32))


# speedup vs baseline: 2.1473x; 2.1473x over previous
import functools

import jax
import jax.numpy as jnp
from jax.experimental import pallas as pl
from jax.experimental.pallas import tpu as pltpu

_SCALE = 64.0
_MARGIN = 0.5
_ROWS = 16


def _arcface_block(x_ref, lab_ref, o_ref):
    x = x_ref[...]
    lab = lab_ref[...]
    valid = lab != -1
    lab_safe = jnp.where(valid, lab, 0)

    inv = jax.lax.rsqrt(jnp.maximum(jnp.sum(x * x, axis=1, keepdims=True),
                                    1e-24))

    cols = jax.lax.broadcasted_iota(jnp.int32, x.shape, 1)
    mask = cols == lab_safe
    t = jnp.sum(jnp.where(mask, x, 0.0), axis=1, keepdims=True) * inv
    t_clip = jnp.clip(t, -1.0, 1.0)
    with_margin = (t_clip * jnp.float32(jnp.cos(_MARGIN))
                   - jnp.float32(jnp.sin(_MARGIN))
                   * jnp.sqrt(jnp.maximum(1.0 - t_clip * t_clip, 0.0)))
    new_val = jnp.where(valid, with_margin, t)

    o_ref[...] = jnp.where(mask, new_val * _SCALE, x * (inv * _SCALE))


@jax.jit
def _run(logits, labels2d):
    n, c = logits.shape
    return pl.pallas_call(
        _arcface_block,
        grid=(n // _ROWS,),
        in_specs=[
            pl.BlockSpec((_ROWS, c), lambda i: (i, 0)),
            pl.BlockSpec((_ROWS, 1), lambda i: (i, 0)),
        ],
        out_specs=pl.BlockSpec((_ROWS, c), lambda i: (i, 0)),
        out_shape=jax.ShapeDtypeStruct((n, c), jnp.float32),
        compiler_params=pltpu.CompilerParams(
            dimension_semantics=("parallel",),
        ),
    )(logits, labels2d)


def kernel(logits, labels):
    labels2d = labels.astype(jnp.int32).reshape(-1, 1)
    return _run(logits, labels2d)
